# baseline (speedup 1.0000x reference)
import functools
import math

import jax
import jax.numpy as jnp
import numpy as np
from jax import lax
from jax.experimental import pallas as pl
from jax.experimental.pallas import tpu as pltpu
from jax.experimental.pallas import tpu_sc as plsc

_SMOOTHING = 0.1
_BN = 32
_NC = 2
_NS = 16
_CW = 2048
_NCH = 24


def _sc_body(eps, coef_g, cs, n_ch, pred_ref, tgt_ref, out_ref,
             buf0, buf1, tgt_v, pv, allv, shared, sem0, sem1):
    core = lax.axis_index("c")
    sid = lax.axis_index("s")
    wid = sid * _NC + core
    rows0 = wid * 32
    lane = lax.iota(jnp.int32, 16)
    zero = jnp.zeros((16,), jnp.float32)
    epsv = jnp.full((16,), eps, jnp.float32)
    coefv = jnp.full((16,), coef_g, jnp.float32)

    pltpu.sync_copy(tgt_ref.at[pl.ds(rows0, 32)], tgt_v)

    total = zero
    bcast_dn = lax.GatherDimensionNumbers(
        offset_dims=(), collapsed_slice_dims=(0,), start_index_map=(0,)
    )

    for g in range(2):
        row0 = rows0 + g * 16
        tv = tgt_v[pl.ds(g * 16, 16)]
        tbm = [
            lax.gather(
                tv, jnp.full((16, 1), r, jnp.int32), bcast_dn, (1,),
                mode=lax.GatherScatterMode.PROMISE_IN_BOUNDS)
            for r in range(16)
        ]

        def _start(c, buf, sem, row0=row0):
            pltpu.make_async_copy(
                pred_ref.at[pl.ds(row0, 16), pl.ds(cs + c * _CW, _CW)],
                buf, sem,
            ).start()

        _start(0, buf0, sem0)
        _start(1, buf1, sem1)

        def _chunk(c, buf, sem, carry, tbm=tbm, row0=row0):
            rs, ga, off_v = carry
            pltpu.make_async_copy(
                pred_ref.at[pl.ds(row0, 16), pl.ds(cs + c * _CW, _CW)],
                buf, sem,
            ).wait()

            def _slice_step(k, st):
                colv = st[0]
                ga_i = st[1]
                accs = st[2:]
                new_accs = []
                for r in range(16):
                    vec = buf[r, pl.ds(k * 16, 16)]
                    new_accs.append(accs[r] + vec)
                    ga_i = ga_i + jnp.where(colv == tbm[r], vec, zero)
                return (colv + 16, ga_i) + tuple(new_accs)

            st = lax.fori_loop(
                0, _CW // 16, _slice_step,
                (off_v + lane, ga) + tuple(zero for _ in range(16)),
            )
            ga = st[1]
            accs = st[2:]
            for r in range(16):
                vm = jnp.minimum(jnp.abs(tbm[r].astype(jnp.float32)), 1.0)
                rs = rs + accs[r] * vm

            @pl.when(c + 2 < n_ch)
            def _():
                _start(c + 2, buf, sem)

            return rs, ga, off_v + _CW

        def _round(k, carry):
            carry = _chunk(2 * k, buf0, sem0, carry)
            carry = _chunk(2 * k + 1, buf1, sem1, carry)
            return carry

        rs, ga, _ = lax.fori_loop(
            0, n_ch // 2, _round,
            (zero, zero, jnp.full((16,), cs, jnp.int32)),
        )
        total = total - epsv * rs - coefv * ga

    pv[...] = total
    pltpu.sync_copy(pv, out_ref.at[wid])


def _sc_partial(pred, target, eps, coef_g, cs, n_ch):
    mesh = plsc.VectorSubcoreMesh(core_axis_name="c", subcore_axis_name="s")
    return pl.kernel(
        functools.partial(_sc_body, eps, coef_g, cs, n_ch),
        out_type=jax.ShapeDtypeStruct((_NC * _NS, 16), jnp.float32),
        mesh=mesh,
        scratch_types=[
            pltpu.VMEM((16, _CW), jnp.float32),
            pltpu.VMEM((16, _CW), jnp.float32),
            pltpu.VMEM((32,), jnp.int32),
            pltpu.VMEM((16,), jnp.float32),
            pltpu.VMEM((_NS, 16), jnp.float32),
            pltpu.VMEM_SHARED((_NS, 16), jnp.float32),
            pltpu.SemaphoreType.DMA,
            pltpu.SemaphoreType.DMA,
        ],
    )(pred, target)


def _tc_body(eps, coef_g, c_row, cs, ce, v, tgt_sref, tgt_ref, pred_ref,
             tail_ref, out_ref):
    i = pl.program_id(0)
    bn = pred_ref.shape[0]
    t = tgt_ref[...]
    x = pred_ref[...]
    tail = tail_ref[...]
    valid = t != 0
    tail_lane = jax.lax.broadcasted_iota(jnp.int32, tail.shape, 1)
    tail_sum = jnp.sum(
        jnp.where(tail_lane < (v - ce), tail, 0.0), axis=1, keepdims=True
    )
    s = jnp.sum(x, axis=1, keepdims=True) - x[:, 0:1] + tail_sum
    part = jnp.sum(jnp.where(valid, s, 0.0))
    cnt = jnp.sum(jnp.where(valid, 1.0, 0.0))

    lane = jax.lax.broadcasted_iota(jnp.int32, (1, 128), 1)
    gpart = jnp.float32(0.0)
    for r in range(bn):
        tr = tgt_sref[i * bn + r]
        start = pl.multiple_of((jnp.minimum(tr, cs - 1) // 128) * 128, 128)
        w = pred_ref[pl.ds(r, 1), pl.ds(start, 128)]
        gval = jnp.sum(jnp.where(lane == tr % 128, w, 0.0))
        gtail = jnp.sum(
            jnp.where(lane == tr - ce, tail_ref[pl.ds(r, 1), :], 0.0)
        )
        gpart += jnp.where((tr != 0) & (tr < cs), gval, 0.0)
        gpart += jnp.where(tr >= ce, gtail, 0.0)

    @pl.when(i == 0)
    def _():
        out_ref[0, 0] = 0.0

    out_ref[0, 0] += c_row * cnt - eps * part - coef_g * gpart


def kernel(pred, target):
    n, v = pred.shape
    ce = (v // 128) * 128
    cs = ce - _NCH * _CW
    eps = _SMOOTHING / (v - 2)
    eps32 = float(np.float32(eps))
    c_row = (v - 2) * (eps32 * math.log(eps32)) + 0.9 * math.log(0.9)
    coef_g = (1.0 - _SMOOTHING) - eps

    sc_out = _sc_partial(pred, target, eps, coef_g, cs, _NCH)

    tgt2d = target.reshape(n, 1)
    grid_spec = pltpu.PrefetchScalarGridSpec(
        num_scalar_prefetch=1,
        grid=(n // _BN,),
        in_specs=[
            pl.BlockSpec((_BN, 1), lambda i, *_: (i, 0)),
            pl.BlockSpec((_BN, cs), lambda i, *_: (i, 0)),
            pl.BlockSpec((_BN, 128), lambda i, *_: (i, ce // 128)),
        ],
        out_specs=pl.BlockSpec(
            (1, 1), lambda i, *_: (0, 0), memory_space=pltpu.SMEM
        ),
    )
    tc_out = pl.pallas_call(
        functools.partial(_tc_body, eps, coef_g, c_row, cs, ce, v),
        grid_spec=grid_spec,
        out_shape=jax.ShapeDtypeStruct((1, 1), jnp.float32),
    )(target, tgt2d, pred, pred)

    return tc_out[0, 0] + jnp.sum(sc_out)


# speedup vs baseline: 1.0151x; 1.0151x over previous
import functools
import math

import jax
import jax.numpy as jnp
import numpy as np
from jax import lax
from jax.experimental import pallas as pl
from jax.experimental.pallas import tpu as pltpu
from jax.experimental.pallas import tpu_sc as plsc

_SMOOTHING = 0.1
_BN = 32
_NC = 2
_NS = 16
_CW = 2048
_NCH = 8


def _sc_body(eps, coef_g, cs, n_ch, pred_ref, tgt_ref, out_ref,
             buf0, buf1, tgt_v, pv, allv, shared, sem0, sem1):
    core = lax.axis_index("c")
    sid = lax.axis_index("s")
    wid = sid * _NC + core
    rows0 = wid * 32
    lane = lax.iota(jnp.int32, 16)
    zero = jnp.zeros((16,), jnp.float32)
    epsv = jnp.full((16,), eps, jnp.float32)
    coefv = jnp.full((16,), coef_g, jnp.float32)

    pltpu.sync_copy(tgt_ref.at[pl.ds(rows0, 32)], tgt_v)

    total = zero
    bcast_dn = lax.GatherDimensionNumbers(
        offset_dims=(), collapsed_slice_dims=(0,), start_index_map=(0,)
    )

    for g in range(2):
        row0 = rows0 + g * 16
        tv = tgt_v[pl.ds(g * 16, 16)]
        tbm = [
            lax.gather(
                tv, jnp.full((16, 1), r, jnp.int32), bcast_dn, (1,),
                mode=lax.GatherScatterMode.PROMISE_IN_BOUNDS)
            for r in range(16)
        ]

        def _start(c, buf, sem, row0=row0):
            pltpu.make_async_copy(
                pred_ref.at[pl.ds(row0, 16), pl.ds(cs + c * _CW, _CW)],
                buf, sem,
            ).start()

        _start(0, buf0, sem0)
        _start(1, buf1, sem1)

        def _chunk(c, buf, sem, carry, tbm=tbm, row0=row0):
            rs, ga, off_v = carry
            pltpu.make_async_copy(
                pred_ref.at[pl.ds(row0, 16), pl.ds(cs + c * _CW, _CW)],
                buf, sem,
            ).wait()

            def _slice_step(k, st):
                colv = st[0]
                ga_i = st[1]
                accs = st[2:]
                new_accs = []
                for r in range(16):
                    vec = buf[r, pl.ds(k * 16, 16)]
                    new_accs.append(accs[r] + vec)
                    ga_i = ga_i + jnp.where(colv == tbm[r], vec, zero)
                return (colv + 16, ga_i) + tuple(new_accs)

            st = lax.fori_loop(
                0, _CW // 16, _slice_step,
                (off_v + lane, ga) + tuple(zero for _ in range(16)),
            )
            ga = st[1]
            accs = st[2:]
            for r in range(16):
                vm = jnp.minimum(jnp.abs(tbm[r].astype(jnp.float32)), 1.0)
                rs = rs + accs[r] * vm

            @pl.when(c + 2 < n_ch)
            def _():
                _start(c + 2, buf, sem)

            return rs, ga, off_v + _CW

        def _round(k, carry):
            carry = _chunk(2 * k, buf0, sem0, carry)
            carry = _chunk(2 * k + 1, buf1, sem1, carry)
            return carry

        rs, ga, _ = lax.fori_loop(
            0, n_ch // 2, _round,
            (zero, zero, jnp.full((16,), cs, jnp.int32)),
        )
        total = total - epsv * rs - coefv * ga

    pv[...] = total
    pltpu.sync_copy(pv, out_ref.at[wid])


def _sc_partial(pred, target, eps, coef_g, cs, n_ch):
    mesh = plsc.VectorSubcoreMesh(core_axis_name="c", subcore_axis_name="s")
    return pl.kernel(
        functools.partial(_sc_body, eps, coef_g, cs, n_ch),
        out_type=jax.ShapeDtypeStruct((_NC * _NS, 16), jnp.float32),
        mesh=mesh,
        scratch_types=[
            pltpu.VMEM((16, _CW), jnp.float32),
            pltpu.VMEM((16, _CW), jnp.float32),
            pltpu.VMEM((32,), jnp.int32),
            pltpu.VMEM((16,), jnp.float32),
            pltpu.VMEM((_NS, 16), jnp.float32),
            pltpu.VMEM_SHARED((_NS, 16), jnp.float32),
            pltpu.SemaphoreType.DMA,
            pltpu.SemaphoreType.DMA,
        ],
    )(pred, target)


def _tc_body(eps, coef_g, c_row, cs, ce, v, tgt_sref, tgt_ref, pred_ref,
             tail_ref, out_ref):
    i = pl.program_id(0)
    bn = pred_ref.shape[0]
    t = tgt_ref[...]
    x = pred_ref[...]
    tail = tail_ref[...]
    valid = t != 0
    tail_lane = jax.lax.broadcasted_iota(jnp.int32, tail.shape, 1)
    tail_sum = jnp.sum(
        jnp.where(tail_lane < (v - ce), tail, 0.0), axis=1, keepdims=True
    )
    s = jnp.sum(x, axis=1, keepdims=True) - x[:, 0:1] + tail_sum
    part = jnp.sum(jnp.where(valid, s, 0.0))
    cnt = jnp.sum(jnp.where(valid, 1.0, 0.0))

    lane = jax.lax.broadcasted_iota(jnp.int32, (1, 128), 1)
    gpart = jnp.float32(0.0)
    for r in range(bn):
        tr = tgt_sref[i * bn + r]
        start = pl.multiple_of((jnp.minimum(tr, cs - 1) // 128) * 128, 128)
        w = pred_ref[pl.ds(r, 1), pl.ds(start, 128)]
        gval = jnp.sum(jnp.where(lane == tr % 128, w, 0.0))
        gtail = jnp.sum(
            jnp.where(lane == tr - ce, tail_ref[pl.ds(r, 1), :], 0.0)
        )
        gpart += jnp.where((tr != 0) & (tr < cs), gval, 0.0)
        gpart += jnp.where(tr >= ce, gtail, 0.0)

    @pl.when(i == 0)
    def _():
        out_ref[0, 0] = 0.0

    out_ref[0, 0] += c_row * cnt - eps * part - coef_g * gpart


def kernel(pred, target):
    n, v = pred.shape
    ce = (v // 128) * 128
    cs = ce - _NCH * _CW
    eps = _SMOOTHING / (v - 2)
    eps32 = float(np.float32(eps))
    c_row = (v - 2) * (eps32 * math.log(eps32)) + 0.9 * math.log(0.9)
    coef_g = (1.0 - _SMOOTHING) - eps

    sc_out = _sc_partial(pred, target, eps, coef_g, cs, _NCH)

    tgt2d = target.reshape(n, 1)
    grid_spec = pltpu.PrefetchScalarGridSpec(
        num_scalar_prefetch=1,
        grid=(n // _BN,),
        in_specs=[
            pl.BlockSpec((_BN, 1), lambda i, *_: (i, 0)),
            pl.BlockSpec((_BN, cs), lambda i, *_: (i, 0)),
            pl.BlockSpec((_BN, 128), lambda i, *_: (i, ce // 128)),
        ],
        out_specs=pl.BlockSpec(
            (1, 1), lambda i, *_: (0, 0), memory_space=pltpu.SMEM
        ),
    )
    tc_out = pl.pallas_call(
        functools.partial(_tc_body, eps, coef_g, c_row, cs, ce, v),
        grid_spec=grid_spec,
        out_shape=jax.ShapeDtypeStruct((1, 1), jnp.float32),
    )(target, tgt2d, pred, pred)

    return tc_out[0, 0] + jnp.sum(sc_out)
